# baseline (speedup 1.0000x reference)
import functools

import jax
import jax.numpy as jnp
from jax import lax
from jax.experimental import pallas as pl
from jax.experimental.pallas import tpu as pltpu
from jax.experimental.pallas import tpu_sc as plsc

_NUM_CORES = 2
_NUM_SUBCORES = 16
_NUM_WORKERS = _NUM_CORES * _NUM_SUBCORES

_CHUNK = 128


@functools.partial(jax.jit, static_argnames=("rows_per_worker",))
def _sc_gather(flat_ids, weight, rows_per_worker):
    n, d = flat_ids.shape[0], weight.shape[1]
    num_chunks = rows_per_worker // _CHUNK

    @functools.partial(
        pl.kernel,
        out_type=jax.ShapeDtypeStruct((n, d), jnp.float32),
        mesh=plsc.VectorSubcoreMesh(core_axis_name="c", subcore_axis_name="s"),
        scratch_types=[
            pltpu.VMEM((_CHUNK,), jnp.int32),
            pltpu.VMEM((_CHUNK, d), jnp.float32),
            pltpu.SemaphoreType.DMA,
        ],
        compiler_params=pltpu.CompilerParams(use_tc_tiling_on_sc=False),
    )
    def k(ids_hbm, table_hbm, out_hbm, idx_v, rows_v, sem):
        wid = lax.axis_index("s") * _NUM_CORES + lax.axis_index("c")
        base = wid * rows_per_worker

        def body(i, carry):
            off = base + i * _CHUNK
            pltpu.sync_copy(ids_hbm.at[pl.ds(off, _CHUNK)], idx_v)
            pltpu.async_copy(table_hbm.at[idx_v], rows_v, sem).wait()
            pltpu.sync_copy(rows_v, out_hbm.at[pl.ds(off, _CHUNK)])
            return carry

        lax.fori_loop(0, num_chunks, body, 0)

    return k(flat_ids, weight)


def kernel(token_ids, weight):
    b, s = token_ids.shape
    d = weight.shape[1]
    flat = token_ids.reshape(-1).astype(jnp.int32)
    n = flat.shape[0]
    rows_per_worker = n // _NUM_WORKERS
    out = _sc_gather(flat, weight, rows_per_worker)
    return out.reshape(b, s, d)


# speedup vs baseline: 1.1907x; 1.1907x over previous
import functools

import jax
import jax.numpy as jnp
from jax import lax
from jax.experimental import pallas as pl
from jax.experimental.pallas import tpu as pltpu
from jax.experimental.pallas import tpu_sc as plsc

_NUM_CORES = 2
_NUM_SUBCORES = 16
_NUM_WORKERS = _NUM_CORES * _NUM_SUBCORES

_BLOCK = 512


@functools.partial(jax.jit, static_argnames=("rows_per_worker",))
def _sc_gather(flat_ids, weight, rows_per_worker):
    n, d = flat_ids.shape[0], weight.shape[1]
    num_blocks = rows_per_worker // _BLOCK

    @functools.partial(
        pl.kernel,
        out_type=jax.ShapeDtypeStruct((n, d), jnp.float32),
        mesh=plsc.VectorSubcoreMesh(core_axis_name="c", subcore_axis_name="s"),
        scratch_types=[
            pltpu.VMEM((rows_per_worker,), jnp.int32),
            pltpu.VMEM((_BLOCK, d), jnp.float32),
            pltpu.VMEM((_BLOCK, d), jnp.float32),
            pltpu.SemaphoreType.DMA,
            pltpu.SemaphoreType.DMA,
            pltpu.SemaphoreType.DMA,
            pltpu.SemaphoreType.DMA,
        ],
        compiler_params=pltpu.CompilerParams(use_tc_tiling_on_sc=False),
    )
    def k(ids_hbm, table_hbm, out_hbm, idx_v, rows0, rows1, sg0, sg1, so0, so1):
        wid = lax.axis_index("s") * _NUM_CORES + lax.axis_index("c")
        base = wid * rows_per_worker
        pltpu.sync_copy(ids_hbm.at[pl.ds(base, rows_per_worker)], idx_v)

        def gather(g, rows, sem):
            pltpu.make_async_copy(
                table_hbm.at[idx_v.at[pl.ds(g * _BLOCK, _BLOCK)]], rows, sem
            ).start()

        def writeback(g, rows, sem):
            pltpu.make_async_copy(
                rows, out_hbm.at[pl.ds(base + g * _BLOCK, _BLOCK)], sem
            ).start()

        def wait(src, dst, sem):
            pltpu.make_async_copy(src, dst, sem).wait()

        gather(0, rows0, sg0)
        gather(1, rows1, sg1)

        def body(it, carry):
            g0 = 2 * it
            g1 = g0 + 1
            wait(table_hbm.at[idx_v.at[pl.ds(0, _BLOCK)]], rows0, sg0)
            writeback(g0, rows0, so0)
            wait(table_hbm.at[idx_v.at[pl.ds(0, _BLOCK)]], rows1, sg1)
            writeback(g1, rows1, so1)

            @pl.when(g0 + 2 < num_blocks)
            def _():
                wait(rows0, out_hbm.at[pl.ds(base, _BLOCK)], so0)
                gather(g0 + 2, rows0, sg0)
                wait(rows1, out_hbm.at[pl.ds(base, _BLOCK)], so1)
                gather(g1 + 2, rows1, sg1)

            return carry

        lax.fori_loop(0, num_blocks // 2, body, 0)

        wait(rows0, out_hbm.at[pl.ds(base, _BLOCK)], so0)
        wait(rows1, out_hbm.at[pl.ds(base, _BLOCK)], so1)

    return k(flat_ids, weight)


def kernel(token_ids, weight):
    b, s = token_ids.shape
    d = weight.shape[1]
    flat = token_ids.reshape(-1).astype(jnp.int32)
    n = flat.shape[0]
    rows_per_worker = n // _NUM_WORKERS
    out = _sc_gather(flat, weight, rows_per_worker)
    return out.reshape(b, s, d)


# speedup vs baseline: 1.5812x; 1.3280x over previous
import functools

import jax
import jax.numpy as jnp
from jax import lax
from jax.experimental import pallas as pl
from jax.experimental.pallas import tpu as pltpu
from jax.experimental.pallas import tpu_sc as plsc

_NUM_CORES = 2
_NUM_SUBCORES = 16
_NUM_WORKERS = _NUM_CORES * _NUM_SUBCORES

_BLOCK = 512


@functools.partial(jax.jit, static_argnames=("rows_per_worker",))
def _sc_gather(flat_ids, weight, rows_per_worker):
    n, d = flat_ids.shape[0], weight.shape[1]
    num_blocks = rows_per_worker // _BLOCK

    @functools.partial(
        pl.kernel,
        out_type=jax.ShapeDtypeStruct((n, 2 * d), jnp.float32),
        mesh=plsc.VectorSubcoreMesh(core_axis_name="c", subcore_axis_name="s"),
        scratch_types=[
            pltpu.VMEM((rows_per_worker,), jnp.int32),
            pltpu.VMEM((_BLOCK, d), jnp.float32),
            pltpu.VMEM((_BLOCK, d), jnp.float32),
            pltpu.SemaphoreType.DMA,
            pltpu.SemaphoreType.DMA,
            pltpu.SemaphoreType.DMA,
            pltpu.SemaphoreType.DMA,
        ],
        compiler_params=pltpu.CompilerParams(use_tc_tiling_on_sc=False),
    )
    def k(ids_hbm, table_hbm, out_hbm, idx_v, rows0, rows1, sg0, sg1, so0, so1):
        wid = lax.axis_index("s") * _NUM_CORES + lax.axis_index("c")
        base = wid * rows_per_worker
        pltpu.sync_copy(ids_hbm.at[pl.ds(base, rows_per_worker)], idx_v)

        def gather(g, rows, sem):
            pltpu.make_async_copy(
                table_hbm.at[idx_v.at[pl.ds(g * _BLOCK, _BLOCK)]], rows, sem
            ).start()

        def writeback(g, rows, sem):
            pltpu.make_async_copy(
                rows,
                out_hbm.at[pl.ds(base + g * _BLOCK, _BLOCK), pl.ds(0, d)],
                sem,
            ).start()

        def wait_gather(rows, sem):
            pltpu.make_async_copy(
                table_hbm.at[idx_v.at[pl.ds(0, _BLOCK)]], rows, sem
            ).wait()

        def wait_writeback(rows, sem):
            pltpu.make_async_copy(
                rows, out_hbm.at[pl.ds(base, _BLOCK), pl.ds(0, d)], sem
            ).wait()

        gather(0, rows0, sg0)
        gather(1, rows1, sg1)

        def body(it, carry):
            g0 = 2 * it
            g1 = g0 + 1
            wait_gather(rows0, sg0)
            writeback(g0, rows0, so0)
            wait_gather(rows1, sg1)
            writeback(g1, rows1, so1)

            @pl.when(g0 + 2 < num_blocks)
            def _():
                wait_writeback(rows0, so0)
                gather(g0 + 2, rows0, sg0)
                wait_writeback(rows1, so1)
                gather(g1 + 2, rows1, sg1)

            return carry

        lax.fori_loop(0, num_blocks // 2, body, 0)

        wait_writeback(rows0, so0)
        wait_writeback(rows1, so1)

    return k(flat_ids, weight)


def kernel(token_ids, weight):
    b, s = token_ids.shape
    d = weight.shape[1]
    flat = token_ids.reshape(-1).astype(jnp.int32)
    n = flat.shape[0]
    rows_per_worker = n // _NUM_WORKERS
    out128 = _sc_gather(flat, weight, rows_per_worker)
    return out128[:, :d].reshape(b, s, d)
